# baseline (speedup 1.0000x reference)
import functools

import jax
import jax.numpy as jnp
from jax import lax
from jax.experimental import pallas as pl
from jax.experimental.pallas import tpu as pltpu
from jax.experimental.pallas import tpu_sc as plsc

N = 10000
E = 320000
D = 128
H = 32
DE = 16

NB = 10
NROW = N // NB
EROW = 4000

NW = 32
EPW = 10048
EPAD = EPW * NW - E
K = 64
NCHUNK = EPW // K
NP = 10240
RPT = NP // 16


def _tc_node_body(x_ref, w1_ref, asr_ref, adr_ref, h_ref, ad_ref):
    h = jnp.dot(x_ref[...], w1_ref[...], preferred_element_type=jnp.float32)
    asv = jnp.sum(h * asr_ref[...], axis=1, keepdims=True)
    h_ref[...] = jnp.concatenate(
        [h, asv, jnp.zeros((h.shape[0], 95), jnp.float32)], axis=1)
    ad_ref[...] = jnp.sum(h * adr_ref[...], axis=1, keepdims=True)


def _tc_node(x, W1, a_src1, a_dst1):
    full = lambda shape: pl.BlockSpec(shape, lambda i: (0,) * len(shape))
    return pl.pallas_call(
        _tc_node_body,
        grid=(NB,),
        in_specs=[
            pl.BlockSpec((NROW, D), lambda i: (i, 0)),
            full((D, H)), full((1, H)), full((1, H)),
        ],
        out_specs=[
            pl.BlockSpec((NROW, 128), lambda i: (i, 0)),
            pl.BlockSpec((NROW, 1), lambda i: (i, 0)),
        ],
        out_shape=[
            jax.ShapeDtypeStruct((N, 128), jnp.float32),
            jax.ShapeDtypeStruct((N, 1), jnp.float32),
        ],
    )(x, W1, a_src1, a_dst1)


def _tc_edge_body(ea_ref, we1_ref, ae1_ref, we2_ref, ae2_ref, e1_ref, e2_ref):
    w1v = jnp.sum(we1_ref[...] * ae1_ref[...], axis=1, keepdims=True)
    w2v = jnp.sum(we2_ref[...] * ae2_ref[...], axis=1, keepdims=True)
    ea = ea_ref[...]
    e1_ref[...] = jnp.dot(ea, w1v, preferred_element_type=jnp.float32)
    e2_ref[...] = jnp.dot(ea, w2v, preferred_element_type=jnp.float32)


def _tc_edge(edge_attr, We1, a_e1, We2, a_e2):
    full = lambda shape: pl.BlockSpec(shape, lambda i: (0,) * len(shape))
    return pl.pallas_call(
        _tc_edge_body,
        grid=(E // EROW,),
        in_specs=[
            pl.BlockSpec((EROW, DE), lambda i: (i, 0)),
            full((DE, H)), full((1, H)), full((DE, H)), full((1, H)),
        ],
        out_specs=[
            pl.BlockSpec((EROW, 1), lambda i: (i, 0)),
            pl.BlockSpec((EROW, 1), lambda i: (i, 0)),
        ],
        out_shape=[
            jax.ShapeDtypeStruct((E, 1), jnp.float32),
            jax.ShapeDtypeStruct((E, 1), jnp.float32),
        ],
    )(edge_attr, We1, a_e1, We2, a_e2)


def _tc_mid_body(acc_ref, b1_ref, w2_ref, asr_ref, adr_ref,
                 h_ref, ad_ref):
    s = acc_ref[0] + acc_ref[1]
    z = s[:, :H] / (s[:, H:H + 1] + 1e-16) + b1_ref[...]
    x2 = jnp.maximum(z, 0.0)
    h = jnp.dot(x2, w2_ref[...], preferred_element_type=jnp.float32)
    asv = jnp.sum(h * asr_ref[...], axis=1, keepdims=True)
    h_ref[...] = jnp.concatenate(
        [h, asv, jnp.zeros((h.shape[0], 95), jnp.float32)], axis=1)
    ad_ref[...] = jnp.sum(h * adr_ref[...], axis=1, keepdims=True)


def _tc_mid(acc, b1, W2, a_src2, a_dst2):
    full = lambda shape: pl.BlockSpec(shape, lambda i: (0,) * len(shape))
    return pl.pallas_call(
        _tc_mid_body,
        grid=(NB,),
        in_specs=[
            pl.BlockSpec((2, NROW, 128), lambda i: (0, i, 0)),
            full((1, H)), full((H, H)), full((1, H)), full((1, H)),
        ],
        out_specs=[
            pl.BlockSpec((NROW, 128), lambda i: (i, 0)),
            pl.BlockSpec((NROW, 1), lambda i: (i, 0)),
        ],
        out_shape=[
            jax.ShapeDtypeStruct((N, 128), jnp.float32),
            jax.ShapeDtypeStruct((N, 1), jnp.float32),
        ],
    )(acc, b1, W2, a_src2, a_dst2)


def _tc_fin_body(acc_ref, b2_ref, z_ref):
    s = acc_ref[0] + acc_ref[1]
    z_ref[...] = s[:, :H] / (s[:, H:H + 1] + 1e-16) + b2_ref[...]


def _tc_fin(acc, b2):
    return pl.pallas_call(
        _tc_fin_body,
        grid=(NB,),
        in_specs=[
            pl.BlockSpec((2, NROW, 128), lambda i: (0, i, 0)),
            pl.BlockSpec((1, H), lambda i: (0, 0)),
        ],
        out_specs=pl.BlockSpec((NROW, H), lambda i: (i, 0)),
        out_shape=jax.ShapeDtypeStruct((N, H), jnp.float32),
    )(acc, b2)


def _sc_body(h_hbm, src_hbm, dst_hbm, ae_hbm, adst_hbm, zeros_hbm,
             out_hbm, adst_v, idxs0, idxs1, idxs2, idxd0, idxd1, idxd2,
             ae0, ae1, ae2, hbuf0, hbuf1, msg0, msg1, p_v, acc_sh,
             sem_g0, sem_g1, sem_i0, sem_i1, sem_i2, sem_s0, sem_s1):
    c = lax.axis_index("c")
    s = lax.axis_index("s")
    wid = s * 2 + c
    rows0 = s * RPT
    idxs = (idxs0, idxs1, idxs2)
    idxd = (idxd0, idxd1, idxd2)
    ae_v = (ae0, ae1, ae2)
    hbuf = (hbuf0, hbuf1)
    msg = (msg0, msg1)
    sem_g = (sem_g0, sem_g1)
    sem_i = (sem_i0, sem_i1, sem_i2)
    sem_s = (sem_s0, sem_s1)
    ebase = pl.multiple_of(wid * EPW, 8)
    pltpu.sync_copy(zeros_hbm.at[pl.ds(rows0, RPT)], acc_sh.at[pl.ds(rows0, RPT)])
    pltpu.sync_copy(zeros_hbm.at[pl.ds(0, K)], msg0)
    pltpu.sync_copy(zeros_hbm.at[pl.ds(0, K)], msg1)
    pltpu.sync_copy(adst_hbm, adst_v)
    plsc.subcore_barrier()

    unit = jnp.where(lax.iota(jnp.int32, 16) == 0,
                     jnp.float32(1.0), jnp.float32(0.0))
    lane = lax.iota(jnp.int32, 16)
    col_as = lane * 0 + H

    def issue_idx(g, i3):
        pltpu.async_copy(src_hbm.at[pl.ds(ebase + g * K, K)], idxs[i3],
                         sem_i[i3])
        pltpu.async_copy(dst_hbm.at[pl.ds(ebase + g * K, K)], idxd[i3],
                         sem_i[i3])
        pltpu.async_copy(ae_hbm.at[pl.ds(ebase + g * K, K)], ae_v[i3],
                         sem_i[i3])

    def wait_idx(g, i3):
        pltpu.make_async_copy(src_hbm.at[pl.ds(ebase + g * K, K)], idxs[i3],
                              sem_i[i3]).wait()
        pltpu.make_async_copy(dst_hbm.at[pl.ds(ebase + g * K, K)], idxd[i3],
                              sem_i[i3]).wait()
        pltpu.make_async_copy(ae_hbm.at[pl.ds(ebase + g * K, K)], ae_v[i3],
                              sem_i[i3]).wait()

    def chunk_step(g, j):
        b = j % 2
        nb = 1 - b
        i3 = j % 3

        @pl.when(g >= 1)
        def _():
            pltpu.make_async_copy(msg[nb], acc_sh.at[idxd[(j - 1) % 3]],
                                  sem_s[nb]).wait()
        @pl.when(g + 1 < NCHUNK)
        def _():
            wait_idx(g + 1, (j + 1) % 3)
            pltpu.async_copy(h_hbm.at[idxs[(j + 1) % 3]], hbuf[nb], sem_g[nb])
        @pl.when(g + 2 < NCHUNK)
        def _():
            issue_idx(g + 2, (j + 2) % 3)

        pltpu.make_async_copy(h_hbm.at[idxs[i3]], hbuf[b], sem_g[b]).wait()

        def alpha_body(gg, carry2):
            i0 = pl.multiple_of(gg * 16, 8)
            a = (plsc.load_gather(hbuf[b], [lane + i0, col_as])
                 + plsc.load_gather(adst_v, [idxd[i3][pl.ds(i0, 16)]])
                 + ae_v[i3][pl.ds(i0, 16)])
            a = jnp.maximum(a, 0.2 * a)
            p_v[pl.ds(i0, 16)] = jnp.exp(a)
            return carry2

        lax.fori_loop(0, K // 16, alpha_body, 0)

        for e in range(K):
            ps = p_v[pl.ds(e, 16)][0]
            msg[b][e, pl.ds(0, 16)] = hbuf[b][e, pl.ds(0, 16)] * ps
            msg[b][e, pl.ds(16, 16)] = hbuf[b][e, pl.ds(16, 16)] * ps
            msg[b][e, pl.ds(32, 16)] = unit * ps
        pltpu.async_copy(msg[b], acc_sh.at[idxd[i3]], sem_s[b], add=True)

    issue_idx(0, 0)
    issue_idx(1, 1)
    wait_idx(0, 0)
    pltpu.async_copy(h_hbm.at[idxs[0]], hbuf[0], sem_g[0])

    def six_body(t, carry):
        g0 = t * 6
        for j in range(6):
            chunk_step(g0 + j, j)
        return carry

    lax.fori_loop(0, NCHUNK // 6, six_body, 0)
    chunk_step(NCHUNK - 1, 0)
    pltpu.make_async_copy(msg[0], acc_sh.at[idxd[0]], sem_s[0]).wait()
    plsc.subcore_barrier()
    pltpu.sync_copy(acc_sh.at[pl.ds(rows0, RPT)],
                    out_hbm.at[c, pl.ds(rows0, RPT)])


_sc_layer = functools.partial(
    pl.kernel,
    mesh=plsc.VectorSubcoreMesh(core_axis_name="c", subcore_axis_name="s"),
    compiler_params=pltpu.CompilerParams(needs_layout_passes=False),
    out_type=jax.ShapeDtypeStruct((2, NP, 128), jnp.float32),
    scratch_types=[
        pltpu.VMEM((NP,), jnp.float32),
        pltpu.VMEM((K,), jnp.int32),
        pltpu.VMEM((K,), jnp.int32),
        pltpu.VMEM((K,), jnp.int32),
        pltpu.VMEM((K,), jnp.int32),
        pltpu.VMEM((K,), jnp.int32),
        pltpu.VMEM((K,), jnp.int32),
        pltpu.VMEM((K,), jnp.float32),
        pltpu.VMEM((K,), jnp.float32),
        pltpu.VMEM((K,), jnp.float32),
        pltpu.VMEM((K, 128), jnp.float32),
        pltpu.VMEM((K, 128), jnp.float32),
        pltpu.VMEM((K, 128), jnp.float32),
        pltpu.VMEM((K, 128), jnp.float32),
        pltpu.VMEM((K + 16,), jnp.float32),
        pltpu.VMEM_SHARED((NP, 128), jnp.float32),
        pltpu.SemaphoreType.DMA,
        pltpu.SemaphoreType.DMA,
        pltpu.SemaphoreType.DMA,
        pltpu.SemaphoreType.DMA,
        pltpu.SemaphoreType.DMA,
        pltpu.SemaphoreType.DMA,
        pltpu.SemaphoreType.DMA,
    ],
)(_sc_body)


@jax.jit
def kernel(x, edge_index, edge_attr, W1, a_src1, a_dst1, We1, a_e1, b1,
           W2, a_src2, a_dst2, We2, a_e2, b2):
    src = jnp.concatenate([edge_index[0].astype(jnp.int32),
                           jnp.zeros((EPAD,), jnp.int32)])
    dst = jnp.concatenate([edge_index[1].astype(jnp.int32),
                           jnp.full((EPAD,), NP - 1, jnp.int32)])
    h1, ad1 = _tc_node(x, W1, a_src1.reshape(1, H), a_dst1.reshape(1, H))
    ae1, ae2 = _tc_edge(edge_attr, We1, a_e1.reshape(1, H),
                        We2, a_e2.reshape(1, H))
    zeros = jnp.zeros((NP, 128), jnp.float32)
    pad_e = jnp.zeros((EPAD,), jnp.float32)
    pad_n = jnp.zeros((NP - N,), jnp.float32)
    acc1 = _sc_layer(h1, src, dst,
                     jnp.concatenate([ae1.reshape(E), pad_e]),
                     jnp.concatenate([ad1.reshape(N), pad_n]), zeros)
    h2, ad2 = _tc_mid(acc1, b1.reshape(1, H), W2,
                      a_src2.reshape(1, H), a_dst2.reshape(1, H))
    acc2 = _sc_layer(h2, src, dst,
                     jnp.concatenate([ae2.reshape(E), pad_e]),
                     jnp.concatenate([ad2.reshape(N), pad_n]), zeros)
    return _tc_fin(acc2, b2.reshape(1, H))


# speedup vs baseline: 1.0762x; 1.0762x over previous
import functools

import jax
import jax.numpy as jnp
from jax import lax
from jax.experimental import pallas as pl
from jax.experimental.pallas import tpu as pltpu
from jax.experimental.pallas import tpu_sc as plsc

N = 10000
E = 320000
D = 128
H = 32
DE = 16

NB = 10
NROW = N // NB
EROW = 4000

NW = 32
EPW = E // NW
K = 80
NCHUNK = EPW // K
NP = 10240
RPT = NP // 16


def _tc_node_body(x_ref, w1_ref, asr_ref, adr_ref, h_ref, ad_ref):
    h = jnp.dot(x_ref[...], w1_ref[...], preferred_element_type=jnp.float32)
    asv = jnp.sum(h * asr_ref[...], axis=1, keepdims=True)
    h_ref[...] = jnp.concatenate(
        [h, asv, jnp.zeros((h.shape[0], 95), jnp.float32)], axis=1)
    ad_ref[...] = jnp.sum(h * adr_ref[...], axis=1, keepdims=True)


def _tc_node(x, W1, a_src1, a_dst1):
    full = lambda shape: pl.BlockSpec(shape, lambda i: (0,) * len(shape))
    return pl.pallas_call(
        _tc_node_body,
        grid=(NB,),
        in_specs=[
            pl.BlockSpec((NROW, D), lambda i: (i, 0)),
            full((D, H)), full((1, H)), full((1, H)),
        ],
        out_specs=[
            pl.BlockSpec((NROW, 128), lambda i: (i, 0)),
            pl.BlockSpec((NROW, 1), lambda i: (i, 0)),
        ],
        out_shape=[
            jax.ShapeDtypeStruct((N, 128), jnp.float32),
            jax.ShapeDtypeStruct((N, 1), jnp.float32),
        ],
    )(x, W1, a_src1, a_dst1)


def _tc_edge_body(ea_ref, we1_ref, ae1_ref, we2_ref, ae2_ref, e1_ref, e2_ref):
    w1v = jnp.sum(we1_ref[...] * ae1_ref[...], axis=1, keepdims=True)
    w2v = jnp.sum(we2_ref[...] * ae2_ref[...], axis=1, keepdims=True)
    ea = ea_ref[...]
    e1_ref[...] = jnp.dot(ea, w1v, preferred_element_type=jnp.float32)
    e2_ref[...] = jnp.dot(ea, w2v, preferred_element_type=jnp.float32)


def _tc_edge(edge_attr, We1, a_e1, We2, a_e2):
    full = lambda shape: pl.BlockSpec(shape, lambda i: (0,) * len(shape))
    return pl.pallas_call(
        _tc_edge_body,
        grid=(E // EROW,),
        in_specs=[
            pl.BlockSpec((EROW, DE), lambda i: (i, 0)),
            full((DE, H)), full((1, H)), full((DE, H)), full((1, H)),
        ],
        out_specs=[
            pl.BlockSpec((EROW, 1), lambda i: (i, 0)),
            pl.BlockSpec((EROW, 1), lambda i: (i, 0)),
        ],
        out_shape=[
            jax.ShapeDtypeStruct((E, 1), jnp.float32),
            jax.ShapeDtypeStruct((E, 1), jnp.float32),
        ],
    )(edge_attr, We1, a_e1, We2, a_e2)


def _tc_mid_body(acc_ref, b1_ref, w2_ref, asr_ref, adr_ref,
                 h_ref, ad_ref):
    s = acc_ref[0] + acc_ref[1]
    z = s[:, :H] / (s[:, H:H + 1] + 1e-16) + b1_ref[...]
    x2 = jnp.maximum(z, 0.0)
    h = jnp.dot(x2, w2_ref[...], preferred_element_type=jnp.float32)
    asv = jnp.sum(h * asr_ref[...], axis=1, keepdims=True)
    h_ref[...] = jnp.concatenate(
        [h, asv, jnp.zeros((h.shape[0], 95), jnp.float32)], axis=1)
    ad_ref[...] = jnp.sum(h * adr_ref[...], axis=1, keepdims=True)


def _tc_mid(acc, b1, W2, a_src2, a_dst2):
    full = lambda shape: pl.BlockSpec(shape, lambda i: (0,) * len(shape))
    return pl.pallas_call(
        _tc_mid_body,
        grid=(NB,),
        in_specs=[
            pl.BlockSpec((2, NROW, 128), lambda i: (0, i, 0)),
            full((1, H)), full((H, H)), full((1, H)), full((1, H)),
        ],
        out_specs=[
            pl.BlockSpec((NROW, 128), lambda i: (i, 0)),
            pl.BlockSpec((NROW, 1), lambda i: (i, 0)),
        ],
        out_shape=[
            jax.ShapeDtypeStruct((N, 128), jnp.float32),
            jax.ShapeDtypeStruct((N, 1), jnp.float32),
        ],
    )(acc, b1, W2, a_src2, a_dst2)


def _tc_fin_body(acc_ref, b2_ref, z_ref):
    s = acc_ref[0] + acc_ref[1]
    z_ref[...] = s[:, :H] / (s[:, H:H + 1] + 1e-16) + b2_ref[...]


def _tc_fin(acc, b2):
    return pl.pallas_call(
        _tc_fin_body,
        grid=(NB,),
        in_specs=[
            pl.BlockSpec((2, NROW, 128), lambda i: (0, i, 0)),
            pl.BlockSpec((1, H), lambda i: (0, 0)),
        ],
        out_specs=pl.BlockSpec((NROW, H), lambda i: (i, 0)),
        out_shape=jax.ShapeDtypeStruct((N, H), jnp.float32),
    )(acc, b2)


def _sc_body(h_hbm, src_hbm, dst_hbm, ae_hbm, adst_hbm, zeros_hbm,
             out_hbm, adst_v, idxs0, idxs1, idxd0, idxd1, ae0, ae1,
             hbuf0, hbuf1, msg_s, p_v, acc_sh,
             sem_g0, sem_g1, sem_i0, sem_i1):
    c = lax.axis_index("c")
    s = lax.axis_index("s")
    wid = s * 2 + c
    rows0 = s * RPT
    idxs = (idxs0, idxs1)
    idxd = (idxd0, idxd1)
    ae_v = (ae0, ae1)
    hbuf = (hbuf0, hbuf1)
    sem_g = (sem_g0, sem_g1)
    sem_i = (sem_i0, sem_i1)
    ebase = pl.multiple_of(wid * EPW, 8)
    pltpu.sync_copy(zeros_hbm.at[pl.ds(rows0, RPT)], acc_sh.at[pl.ds(rows0, RPT)])
    pltpu.sync_copy(zeros_hbm.at[pl.ds(0, K)], msg_s)
    pltpu.sync_copy(adst_hbm, adst_v)
    plsc.subcore_barrier()

    unit = jnp.where(lax.iota(jnp.int32, 16) == 0,
                     jnp.float32(1.0), jnp.float32(0.0))
    lane = lax.iota(jnp.int32, 16)
    col_as = lane * 0 + H

    def issue_idx(g, b):
        pltpu.async_copy(src_hbm.at[pl.ds(ebase + g * K, K)], idxs[b], sem_i[b])
        pltpu.async_copy(dst_hbm.at[pl.ds(ebase + g * K, K)], idxd[b], sem_i[b])
        pltpu.async_copy(ae_hbm.at[pl.ds(ebase + g * K, K)], ae_v[b], sem_i[b])

    def wait_idx(g, b):
        pltpu.make_async_copy(src_hbm.at[pl.ds(ebase + g * K, K)], idxs[b],
                              sem_i[b]).wait()
        pltpu.make_async_copy(dst_hbm.at[pl.ds(ebase + g * K, K)], idxd[b],
                              sem_i[b]).wait()
        pltpu.make_async_copy(ae_hbm.at[pl.ds(ebase + g * K, K)], ae_v[b],
                              sem_i[b]).wait()

    def chunk_step(g, b):
        nb = 1 - b

        @pl.when(g + 1 < NCHUNK)
        def _():
            wait_idx(g + 1, nb)
            pltpu.async_copy(h_hbm.at[idxs[nb]], hbuf[nb], sem_g[nb])

        pltpu.make_async_copy(h_hbm.at[idxs[b]], hbuf[b], sem_g[b]).wait()

        def alpha_body(gg, carry2):
            i0 = pl.multiple_of(gg * 16, 8)
            a = (plsc.load_gather(hbuf[b], [lane + i0, col_as])
                 + plsc.load_gather(adst_v, [idxd[b][pl.ds(i0, 16)]])
                 + ae_v[b][pl.ds(i0, 16)])
            a = jnp.maximum(a, 0.2 * a)
            p_v[pl.ds(i0, 16)] = jnp.exp(a)
            return carry2

        lax.fori_loop(0, K // 16, alpha_body, 0)

        for e in range(K):
            ps = p_v[pl.ds(e, 16)][0]
            msg_s[e, pl.ds(0, 16)] = hbuf[b][e, pl.ds(0, 16)] * ps
            msg_s[e, pl.ds(16, 16)] = hbuf[b][e, pl.ds(16, 16)] * ps
            msg_s[e, pl.ds(32, 16)] = unit * ps
        pltpu.sync_copy(msg_s, acc_sh.at[idxd[b]], add=True)

        @pl.when(g + 2 < NCHUNK)
        def _():
            issue_idx(g + 2, b)

    issue_idx(0, 0)
    wait_idx(0, 0)
    pltpu.async_copy(h_hbm.at[idxs[0]], hbuf[0], sem_g[0])
    issue_idx(1, 1)

    def pair_body(t, carry):
        g0 = t * 2
        chunk_step(g0, 0)
        chunk_step(g0 + 1, 1)
        return carry

    lax.fori_loop(0, (NCHUNK - 1) // 2, pair_body, 0)
    chunk_step(NCHUNK - 1, 0)
    plsc.subcore_barrier()
    pltpu.sync_copy(acc_sh.at[pl.ds(rows0, RPT)],
                    out_hbm.at[c, pl.ds(rows0, RPT)])


_sc_layer = functools.partial(
    pl.kernel,
    mesh=plsc.VectorSubcoreMesh(core_axis_name="c", subcore_axis_name="s"),
    compiler_params=pltpu.CompilerParams(needs_layout_passes=False),
    out_type=jax.ShapeDtypeStruct((2, NP, 128), jnp.float32),
    scratch_types=[
        pltpu.VMEM((N,), jnp.float32),
        pltpu.VMEM((K,), jnp.int32),
        pltpu.VMEM((K,), jnp.int32),
        pltpu.VMEM((K,), jnp.int32),
        pltpu.VMEM((K,), jnp.int32),
        pltpu.VMEM((K,), jnp.float32),
        pltpu.VMEM((K,), jnp.float32),
        pltpu.VMEM((K, 128), jnp.float32),
        pltpu.VMEM((K, 128), jnp.float32),
        pltpu.VMEM((K, 128), jnp.float32),
        pltpu.VMEM((K + 16,), jnp.float32),
        pltpu.VMEM_SHARED((NP, 128), jnp.float32),
        pltpu.SemaphoreType.DMA,
        pltpu.SemaphoreType.DMA,
        pltpu.SemaphoreType.DMA,
        pltpu.SemaphoreType.DMA,
    ],
)(_sc_body)


@jax.jit
def kernel(x, edge_index, edge_attr, W1, a_src1, a_dst1, We1, a_e1, b1,
           W2, a_src2, a_dst2, We2, a_e2, b2):
    src = edge_index[0].astype(jnp.int32)
    dst = edge_index[1].astype(jnp.int32)
    h1, ad1 = _tc_node(x, W1, a_src1.reshape(1, H), a_dst1.reshape(1, H))
    ae1, ae2 = _tc_edge(edge_attr, We1, a_e1.reshape(1, H),
                        We2, a_e2.reshape(1, H))
    zeros = jnp.zeros((NP, 128), jnp.float32)
    acc1 = _sc_layer(h1, src, dst, ae1.reshape(E), ad1.reshape(N), zeros)
    h2, ad2 = _tc_mid(acc1, b1.reshape(1, H), W2,
                      a_src2.reshape(1, H), a_dst2.reshape(1, H))
    acc2 = _sc_layer(h2, src, dst, ae2.reshape(E), ad2.reshape(N), zeros)
    return _tc_fin(acc2, b2.reshape(1, H))
